# baseline (device time: 64937 ns/iter reference)
import jax
import jax.numpy as jnp
from jax import lax
from jax.experimental import pallas as pl
from jax.experimental.pallas import tpu as pltpu

N_DEV = 8
N_TOK = 2048
D_MODEL = 512
D_HID = 1024
N_EXP = 32
N_EXP_LOCAL = N_EXP // N_DEV
CHUNK = N_TOK // N_DEV
HALF = CHUNK // 2
DIR_SPLIT = 4
SUBH = HALF // DIR_SPLIT
SUBRINGS = [(d, h) for h in range(DIR_SPLIT) for d in (0, 1)]
N_SLOTS = 2 * (N_DEV - 1) + 1


def kernel(x, router_W, route_idx, expert_W):
    def body(x_ref, rw_ref, idx_ref, ew_ref, out_ref, scores_ref, cb_ref,
             comm_ref, send_sems, recv_sems):
        my = lax.axis_index("i")
        left = lax.rem(my + N_DEV - 1, N_DEV)
        right = lax.rem(my + 1, N_DEV)

        barrier_sem = pltpu.get_barrier_semaphore()
        for nbr in (left, right):
            pl.semaphore_signal(
                barrier_sem, inc=1,
                device_id=(nbr,), device_id_type=pl.DeviceIdType.MESH,
            )
        pl.semaphore_wait(barrier_sem, 2)

        scores_ref[:, :] = jnp.dot(x_ref[:, :], rw_ref[:, :],
                                   preferred_element_type=jnp.float32)
        ewb = ew_ref[:, :, :].astype(jnp.bfloat16).reshape(
            N_EXP_LOCAL * D_MODEL, D_HID)

        def compute_half(c, d):
            r0 = c * CHUNK + d * HALF
            sc = scores_ref[pl.ds(r0, HALF), :]
            idxc = idx_ref[pl.ds(r0, HALF), :]
            e0 = idxc[:, 0:1]
            e1 = idxc[:, 1:2]
            iota = lax.broadcasted_iota(jnp.int32, (HALF, N_EXP), 1)
            s0 = jnp.sum(jnp.where(iota == e0, sc, 0.0), axis=1, keepdims=True)
            s1 = jnp.sum(jnp.where(iota == e1, sc, 0.0), axis=1, keepdims=True)
            m = jnp.maximum(s0, s1)
            g0 = jnp.exp(s0 - m)
            g1 = jnp.exp(s1 - m)
            w0 = g0 / (g0 + g1)
            w1 = g1 / (g0 + g1)
            xc = x_ref[pl.ds(r0, HALF), :].astype(jnp.bfloat16)
            parts = []
            for k in range(N_EXP_LOCAL):
                ge = my * N_EXP_LOCAL + k
                gate = (w0 * (e0 == ge).astype(jnp.float32)
                        + w1 * (e1 == ge).astype(jnp.float32))
                parts.append(xc * gate.astype(jnp.bfloat16))
            xg = jnp.concatenate(parts, axis=1)
            y = jnp.dot(xg, ewb, preferred_element_type=jnp.float32)
            cb_ref[pl.ds(d * HALF, HALF), :] = y.astype(jnp.bfloat16)

        def cb_piece(d, h):
            return pl.ds(d * HALF + h * SUBH, SUBH)

        def hop(i, s):
            d, _ = SUBRINGS[i]
            return pltpu.make_async_remote_copy(
                src_ref=comm_ref.at[i, s],
                dst_ref=comm_ref.at[i, s + 1],
                send_sem=send_sems.at[i, s],
                recv_sem=recv_sems.at[i, s + 1],
                device_id=(right,) if d == 0 else (left,),
                device_id_type=pl.DeviceIdType.MESH,
            )

        rs_chunk = (
            lambda s: lax.rem(my + N_DEV - s - 1, N_DEV),
            lambda s: lax.rem(my + s + 1, N_DEV),
        )

        cur = [None] * len(SUBRINGS)
        for d in (0, 1):
            compute_half(my, d)
            for i, (dd, h) in enumerate(SUBRINGS):
                if dd == d:
                    comm_ref[i, 0] = cb_ref[cb_piece(d, h), :]
                    cur[i] = hop(i, 0)
                    cur[i].start()
        for d in (0, 1):
            compute_half(rs_chunk[d](0), d)
        for s in range(N_DEV - 1):
            last = s == N_DEV - 2
            nxt = [hop(i, s + 1) for i in range(len(SUBRINGS))]
            for i, (d, h) in enumerate(SUBRINGS):
                cur[i].wait_recv()
                comm_ref[i, s + 1] = (comm_ref[i, s + 1][:, :]
                                      + cb_ref[cb_piece(d, h), :])
                nxt[i].start()
            if not last:
                for d in (0, 1):
                    compute_half(rs_chunk[d](s + 1), d)
            cur = nxt

        ag_chunk = (
            lambda s: lax.rem(my + N_DEV - s, N_DEV),
            lambda s: lax.rem(my + s, N_DEV),
        )
        own = (lax.rem(my + 1, N_DEV), lax.rem(my + N_DEV - 1, N_DEV))
        for i, (d, h) in enumerate(SUBRINGS):
            out_ref[pl.ds(own[d] * CHUNK + d * HALF + h * SUBH, SUBH), :] = (
                comm_ref[i, N_DEV - 1][:, :])
        for s in range(N_DEV - 1):
            slot = N_DEV + s
            last = s == N_DEV - 2
            nxt = None if last else [hop(i, slot) for i in range(len(SUBRINGS))]
            for i, (d, h) in enumerate(SUBRINGS):
                cur[i].wait_recv()
                if nxt is not None:
                    nxt[i].start()
                out_ref[pl.ds(ag_chunk[d](s) * CHUNK + d * HALF + h * SUBH,
                              SUBH), :] = comm_ref[i, slot][:, :]
            if nxt is not None:
                cur = nxt

        for s in range(2 * (N_DEV - 1)):
            for i in range(len(SUBRINGS)):
                hop(i, s).wait_send()

    return pl.pallas_call(
        body,
        out_shape=jax.ShapeDtypeStruct((N_TOK, D_HID), jnp.bfloat16),
        in_specs=[
            pl.BlockSpec(memory_space=pltpu.VMEM),
            pl.BlockSpec(memory_space=pltpu.VMEM),
            pl.BlockSpec(memory_space=pltpu.VMEM),
            pl.BlockSpec(memory_space=pltpu.VMEM),
        ],
        out_specs=pl.BlockSpec(memory_space=pltpu.VMEM),
        scratch_shapes=[
            pltpu.VMEM((N_TOK, N_EXP), jnp.float32),
            pltpu.VMEM((CHUNK, D_HID), jnp.bfloat16),
            pltpu.VMEM((len(SUBRINGS), N_SLOTS, SUBH, D_HID),
                       jnp.bfloat16),
            pltpu.SemaphoreType.DMA((len(SUBRINGS), N_SLOTS)),
            pltpu.SemaphoreType.DMA((len(SUBRINGS), N_SLOTS)),
        ],
        compiler_params=pltpu.CompilerParams(collective_id=0),
    )(x, router_W, route_idx, expert_W)


# device time: 64043 ns/iter; 1.0140x vs baseline; 1.0140x over previous
import jax
import jax.numpy as jnp
from jax import lax
from jax.experimental import pallas as pl
from jax.experimental.pallas import tpu as pltpu

N_DEV = 8
P_RING = 4
N_TOK = 2048
D_MODEL = 512
D_HID = 1024
N_EXP = 32
N_EXP_LOCAL = N_EXP // N_DEV
CHUNK = N_TOK // N_DEV
QUARTER = 2 * CHUNK
PIECES = 2
PSUB = CHUNK // PIECES
SUBRINGS = [(d, h) for h in range(PIECES) for d in (0, 1)]
N_RINGS = len(SUBRINGS)


def kernel(x, router_W, route_idx, expert_W):
    def body(x_ref, rw_ref, idx_ref, ew_ref, out_ref, scores_ref, cb_ref,
             comm_ref, zbuf_ref, rs_send, rs_recv, z1_send, z1_recv,
             z2_send, z2_recv, ag_send_sems, ag_recv_sems):
        my = lax.axis_index("i")
        p = my // P_RING
        q = lax.rem(my, P_RING)
        right = p * P_RING + lax.rem(q + 1, P_RING)
        left = p * P_RING + lax.rem(q + P_RING - 1, P_RING)
        mirror = lax.rem(my + P_RING, N_DEV)

        barrier_sem = pltpu.get_barrier_semaphore()
        for nbr in (left, right, mirror):
            pl.semaphore_signal(
                barrier_sem, inc=1,
                device_id=(nbr,), device_id_type=pl.DeviceIdType.MESH,
            )
        pl.semaphore_wait(barrier_sem, 3)

        scores_ref[:, :] = jnp.dot(x_ref[:, :], rw_ref[:, :],
                                   preferred_element_type=jnp.float32)
        ewb = ew_ref[:, :, :].astype(jnp.bfloat16).reshape(
            N_EXP_LOCAL * D_MODEL, D_HID)

        def compute_chunk(c, slot):
            r0 = c * CHUNK
            sc = scores_ref[pl.ds(r0, CHUNK), :]
            idxc = idx_ref[pl.ds(r0, CHUNK), :]
            e0 = idxc[:, 0:1]
            e1 = idxc[:, 1:2]
            iota = lax.broadcasted_iota(jnp.int32, (CHUNK, N_EXP), 1)
            s0 = jnp.sum(jnp.where(iota == e0, sc, 0.0), axis=1, keepdims=True)
            s1 = jnp.sum(jnp.where(iota == e1, sc, 0.0), axis=1, keepdims=True)
            m = jnp.maximum(s0, s1)
            g0 = jnp.exp(s0 - m)
            g1 = jnp.exp(s1 - m)
            w0 = g0 / (g0 + g1)
            w1 = g1 / (g0 + g1)
            xc = x_ref[pl.ds(r0, CHUNK), :].astype(jnp.bfloat16)
            parts = []
            for k in range(N_EXP_LOCAL):
                ge = my * N_EXP_LOCAL + k
                gate = (w0 * (e0 == ge).astype(jnp.float32)
                        + w1 * (e1 == ge).astype(jnp.float32))
                parts.append(xc * gate.astype(jnp.bfloat16))
            xg = jnp.concatenate(parts, axis=1)
            y = jnp.dot(xg, ewb, preferred_element_type=jnp.float32)
            cb_ref[pl.ds(slot * CHUNK, CHUNK), :] = y.astype(jnp.bfloat16)

        def ring_i(d, h):
            return SUBRINGS.index((d, h))

        def rs_hop(i, s):
            d, _ = SUBRINGS[i]
            return pltpu.make_async_remote_copy(
                src_ref=comm_ref.at[i, s],
                dst_ref=comm_ref.at[i, s + 1],
                send_sem=rs_send.at[i, s],
                recv_sem=rs_recv.at[i, s + 1],
                device_id=(right,) if d == 0 else (left,),
                device_id_type=pl.DeviceIdType.MESH,
            )

        acc_chunk = (
            lambda s: 2 * lax.rem(q + P_RING - s - 1, P_RING),
            lambda s: 2 * lax.rem(q + s + 3, P_RING) + 1,
        )
        first_chunk = (2 * q, 2 * lax.rem(q + 2, P_RING) + 1)

        cur = [None] * N_RINGS
        for d in (0, 1):
            compute_chunk(first_chunk[d], d)
            for h in range(PIECES):
                i = ring_i(d, h)
                comm_ref[i, 0] = cb_ref[pl.ds(d * CHUNK + h * PSUB, PSUB), :]
                cur[i] = rs_hop(i, 0)
                cur[i].start()
        for d in (0, 1):
            compute_chunk(acc_chunk[d](0), d)

        qown = lax.rem(q + 1, P_RING)
        c_own = 2 * qown + p
        c_mir = 2 * qown + (1 - p)

        def z1_desc(h):
            src_i = (1 - p) + 2 * h
            return pltpu.make_async_remote_copy(
                src_ref=comm_ref.at[src_i, P_RING - 1],
                dst_ref=zbuf_ref.at[h],
                send_sem=z1_send.at[h],
                recv_sem=z1_recv.at[h],
                device_id=(mirror,),
                device_id_type=pl.DeviceIdType.MESH,
            )

        for s in range(P_RING - 1):
            last = s == P_RING - 2
            nxt = None if last else [rs_hop(i, s + 1) for i in range(N_RINGS)]
            for i, (d, h) in enumerate(SUBRINGS):
                cur[i].wait_recv()
                comm_ref[i, s + 1] = (
                    comm_ref[i, s + 1][:, :]
                    + cb_ref[pl.ds(d * CHUNK + h * PSUB, PSUB), :])
                if nxt is not None:
                    nxt[i].start()
                elif d == 1:
                    z1_desc(h).start()
            if not last:
                for d in (0, 1):
                    compute_chunk(acc_chunk[d](s + 1), d)
                cur = nxt

        def out_rows(c, h):
            return pl.ds(c * CHUNK + h * PSUB, PSUB)

        def z2_desc(h):
            return pltpu.make_async_remote_copy(
                src_ref=out_ref.at[out_rows(c_own, h), :],
                dst_ref=out_ref.at[out_rows(c_own, h), :],
                send_sem=z2_send.at[h],
                recv_sem=z2_recv.at[h],
                device_id=(mirror,),
                device_id_type=pl.DeviceIdType.MESH,
            )

        def z2_recv_desc(h):
            return pltpu.make_async_remote_copy(
                src_ref=out_ref.at[out_rows(c_mir, h), :],
                dst_ref=out_ref.at[out_rows(c_mir, h), :],
                send_sem=z2_send.at[h],
                recv_sem=z2_recv.at[h],
                device_id=(mirror,),
                device_id_type=pl.DeviceIdType.MESH,
            )

        keep_i = (lambda h: p + 2 * h)
        for h in range(PIECES):
            z1_desc(h).wait_recv()
            out_ref[out_rows(c_own, h), :] = (
                comm_ref[keep_i(h), P_RING - 1][:, :] + zbuf_ref[h][:, :])
            z2_desc(h).start()

        ag_sent = (
            lambda s: 2 * lax.rem(q + P_RING + 1 - s, P_RING),
            lambda s: 2 * lax.rem(q + 1 + s, P_RING) + 1,
        )
        ag_rcvd = (
            lambda s: 2 * lax.rem(q + P_RING - s, P_RING),
            lambda s: 2 * lax.rem(q + 2 + s, P_RING) + 1,
        )

        def ag_send(i, s):
            d, h = SUBRINGS[i]
            rows = out_rows(ag_sent[d](s), h)
            return pltpu.make_async_remote_copy(
                src_ref=out_ref.at[rows, :],
                dst_ref=out_ref.at[rows, :],
                send_sem=ag_send_sems.at[i, s],
                recv_sem=ag_recv_sems.at[i, s],
                device_id=(right,) if d == 0 else (left,),
                device_id_type=pl.DeviceIdType.MESH,
            )

        def ag_recv(i, s):
            d, h = SUBRINGS[i]
            rows = out_rows(ag_rcvd[d](s), h)
            return pltpu.make_async_remote_copy(
                src_ref=out_ref.at[rows, :],
                dst_ref=out_ref.at[rows, :],
                send_sem=ag_send_sems.at[i, s],
                recv_sem=ag_recv_sems.at[i, s],
                device_id=(right,) if d == 0 else (left,),
                device_id_type=pl.DeviceIdType.MESH,
            )

        for h in range(PIECES):
            z2_recv_desc(h).wait_recv()
            for d in (0, 1):
                ag_send(ring_i(d, h), 0).start()
        for s in range(P_RING - 1):
            for i in range(N_RINGS):
                ag_recv(i, s).wait_recv()
                if s < P_RING - 2:
                    ag_send(i, s + 1).start()

        for i in range(N_RINGS):
            for s in range(P_RING - 1):
                rs_hop(i, s).wait_send()
                ag_send(i, s).wait_send()
        for h in range(PIECES):
            z1_desc(h).wait_send()
            z2_desc(h).wait_send()

    return pl.pallas_call(
        body,
        out_shape=jax.ShapeDtypeStruct((N_TOK, D_HID), jnp.bfloat16),
        in_specs=[
            pl.BlockSpec(memory_space=pltpu.VMEM),
            pl.BlockSpec(memory_space=pltpu.VMEM),
            pl.BlockSpec(memory_space=pltpu.VMEM),
            pl.BlockSpec(memory_space=pltpu.VMEM),
        ],
        out_specs=pl.BlockSpec(memory_space=pltpu.VMEM),
        scratch_shapes=[
            pltpu.VMEM((N_TOK, N_EXP), jnp.float32),
            pltpu.VMEM((2 * CHUNK, D_HID), jnp.bfloat16),
            pltpu.VMEM((N_RINGS, P_RING, PSUB, D_HID),
                       jnp.bfloat16),
            pltpu.VMEM((PIECES, PSUB, D_HID), jnp.bfloat16),
            pltpu.SemaphoreType.DMA((N_RINGS, P_RING)),
            pltpu.SemaphoreType.DMA((N_RINGS, P_RING)),
            pltpu.SemaphoreType.DMA((PIECES,)),
            pltpu.SemaphoreType.DMA((PIECES,)),
            pltpu.SemaphoreType.DMA((PIECES,)),
            pltpu.SemaphoreType.DMA((PIECES,)),
            pltpu.SemaphoreType.DMA((N_RINGS, P_RING - 1)),
            pltpu.SemaphoreType.DMA((N_RINGS, P_RING - 1)),
        ],
        compiler_params=pltpu.CompilerParams(collective_id=0),
    )(x, router_W, route_idx, expert_W)
